# baseline (speedup 1.0000x reference)
import functools

import jax
import jax.numpy as jnp
from jax import lax
from jax.experimental import pallas as pl
from jax.experimental.pallas import tpu as pltpu
from jax.experimental.pallas import tpu_sc as plsc


def _sc_gather(E, idx_flat):
    N = idx_flat.shape[0]
    D = E.shape[1]
    info = plsc.get_sparse_core_info()
    NC = info.num_cores
    NW = NC * info.num_subcores
    b_per_w = N // NW
    CH = 400
    n_chunks = b_per_w // CH
    assert N % NW == 0 and b_per_w % CH == 0 and CH % 16 == 0
    mesh = plsc.VectorSubcoreMesh(core_axis_name="c", subcore_axis_name="s")

    @functools.partial(
        pl.kernel,
        mesh=mesh,
        out_type=jax.ShapeDtypeStruct((N, D), jnp.float32),
        scratch_types=[
            pltpu.VMEM((CH,), jnp.int32),
            pltpu.VMEM((CH, D), jnp.float32),
            pltpu.SemaphoreType.DMA,
        ],
    )
    def gather_k(table_hbm, idx_hbm, out_hbm, idx_v, rows_v, sem):
        wid = lax.axis_index("s") * NC + lax.axis_index("c")
        base = wid * b_per_w

        def one_chunk(cbase):
            pltpu.sync_copy(idx_hbm.at[pl.ds(cbase, CH)], idx_v)

            @pl.loop(0, CH // 16)
            def _issue(g):
                v = idx_v[pl.ds(g * 16, 16)]
                for j in range(16):
                    pltpu.async_copy(
                        table_hbm.at[pl.ds(v[j], 1)],
                        rows_v.at[pl.ds(g * 16 + j, 1)],
                        sem,
                    )

                @pl.when(g > 0)
                def _():
                    for _ in range(16):
                        pltpu.make_async_copy(
                            table_hbm.at[pl.ds(0, 1)], rows_v.at[pl.ds(0, 1)], sem
                        ).wait()

            for _ in range(16):
                pltpu.make_async_copy(
                    table_hbm.at[pl.ds(0, 1)], rows_v.at[pl.ds(0, 1)], sem
                ).wait()

            pltpu.sync_copy(rows_v, out_hbm.at[pl.ds(cbase, CH)])

        for c in range(n_chunks):
            one_chunk(base + c * CH)

    return gather_k(E, idx_flat)


def _gru_t_body(emb_ref, x_ref, wie_ref, wio_ref, whh_ref, bih_ref, bhh_ref,
                outt_ref):
    Lx = emb_ref.shape[0]
    Hd = whh_ref.shape[0]
    Bc = emb_ref.shape[1]
    wie = wie_ref[...]
    wio = wio_ref[...]
    whh = whh_ref[...]
    bih = bih_ref[...]
    bhh = bhh_ref[...]

    def step(t, h):
        xt = emb_ref[t]
        par = (x_ref[t] & 1).reshape(1, Bc)
        gi_e = lax.dot_general(wie, xt, (((0,), (1,)), ((), ())),
                               preferred_element_type=jnp.float32)
        gi_o = lax.dot_general(wio, xt, (((0,), (1,)), ((), ())),
                               preferred_element_type=jnp.float32)
        gi = jnp.where(par == 1, gi_o, gi_e) + bih
        gh = lax.dot_general(whh, h, (((0,), (0,)), ((), ())),
                             preferred_element_type=jnp.float32) + bhh
        i_r, i_z, i_n = gi[:Hd], gi[Hd:2 * Hd], gi[2 * Hd:]
        h_r, h_z, h_n = gh[:Hd], gh[Hd:2 * Hd], gh[2 * Hd:]
        r = jax.nn.sigmoid(i_r + h_r)
        z = jax.nn.sigmoid(i_z + h_z)
        n = jnp.tanh(i_n + r * h_n)
        h_new = (1.0 - z) * n + z * h
        outt_ref[t] = h_new
        return h_new

    h = jnp.zeros((Hd, Bc), jnp.float32)
    for t in range(Lx):
        h = step(t, h)


def _gru_t(emb, x, W_ih, W_hh, b_ih, b_hh, interpret=False):
    Lx, Bx, H2 = emb.shape
    Hx = W_ih.shape[1]
    wih_t = W_ih.T
    zeros = jnp.zeros_like(wih_t)
    wie = jnp.concatenate([wih_t, zeros], axis=0)
    wio = jnp.concatenate([zeros, wih_t], axis=0)
    whh_t = W_hh.T
    bih2 = b_ih.reshape(3 * Hx, 1)
    bhh2 = b_hh.reshape(3 * Hx, 1)
    return pl.pallas_call(
        _gru_t_body,
        grid=(1,),
        in_specs=[
            pl.BlockSpec((Lx, Bx, H2), lambda i: (0, 0, 0)),
            pl.BlockSpec((Lx, Bx), lambda i: (0, 0)),
            pl.BlockSpec((H2, 3 * Hx), lambda i: (0, 0)),
            pl.BlockSpec((H2, 3 * Hx), lambda i: (0, 0)),
            pl.BlockSpec((Hx, 3 * Hx), lambda i: (0, 0)),
            pl.BlockSpec((3 * Hx, 1), lambda i: (0, 0)),
            pl.BlockSpec((3 * Hx, 1), lambda i: (0, 0)),
        ],
        out_specs=pl.BlockSpec((Lx, Hx, Bx), lambda i: (0, 0, 0)),
        out_shape=jax.ShapeDtypeStruct((Lx, Hx, Bx), jnp.float32),
        interpret=interpret,
    )(emb, x, wie, wio, whh_t, bih2, bhh2)


def kernel(x, E, W_ih, W_hh, b_ih, b_hh):
    Lx, Bx = x.shape
    Hx = E.shape[1]
    E2 = E.reshape(E.shape[0] // 2, 2 * Hx)
    emb2 = _sc_gather(E2, (x >> 1).reshape(-1)).reshape(Lx, Bx, 2 * Hx)
    out_t = _gru_t(emb2, x, W_ih, W_hh, b_ih, b_hh)
    out = jnp.transpose(out_t, (0, 2, 1))
    return out, out[Lx - 1:Lx]


# speedup vs baseline: 1.6553x; 1.6553x over previous
import functools

import jax
import jax.numpy as jnp
from jax import lax
from jax.experimental import pallas as pl
from jax.experimental.pallas import tpu as pltpu
from jax.experimental.pallas import tpu_sc as plsc


def _sc_gather(E, idx_flat):
    N = idx_flat.shape[0]
    D = E.shape[1]
    info = plsc.get_sparse_core_info()
    NC = info.num_cores
    NW = NC * info.num_subcores
    b_per_w = N // NW
    CH = 400
    n_chunks = b_per_w // CH
    assert N % NW == 0 and b_per_w % CH == 0 and CH % 16 == 0
    mesh = plsc.VectorSubcoreMesh(core_axis_name="c", subcore_axis_name="s")

    @functools.partial(
        pl.kernel,
        mesh=mesh,
        out_type=jax.ShapeDtypeStruct((N, D), jnp.float32),
        scratch_types=[
            pltpu.VMEM((CH,), jnp.int32),
            pltpu.VMEM((CH, D), jnp.float32),
            pltpu.SemaphoreType.DMA,
        ],
    )
    def gather_k(table_hbm, idx_hbm, out_hbm, idx_v, rows_v, sem):
        wid = lax.axis_index("s") * NC + lax.axis_index("c")
        base = wid * b_per_w

        def one_chunk(cbase):
            pltpu.sync_copy(idx_hbm.at[pl.ds(cbase, CH)], idx_v)

            @pl.loop(0, CH // 16)
            def _issue(g):
                v = idx_v[pl.ds(g * 16, 16)]
                for j in range(16):
                    pltpu.async_copy(
                        table_hbm.at[pl.ds(v[j], 1)],
                        rows_v.at[pl.ds(g * 16 + j, 1)],
                        sem,
                    )

                @pl.when(g > 1)
                def _():
                    for _ in range(16):
                        pltpu.make_async_copy(
                            table_hbm.at[pl.ds(0, 1)], rows_v.at[pl.ds(0, 1)], sem
                        ).wait()

            for _ in range(32):
                pltpu.make_async_copy(
                    table_hbm.at[pl.ds(0, 1)], rows_v.at[pl.ds(0, 1)], sem
                ).wait()

            pltpu.sync_copy(rows_v, out_hbm.at[pl.ds(cbase, CH)])

        for c in range(n_chunks):
            one_chunk(base + c * CH)

    return gather_k(E, idx_flat)


def _gru_t_body(emb_ref, wih_ref, whh_ref, bih_ref, bhh_ref, outt_ref):
    Lx = emb_ref.shape[0]
    Hd = wih_ref.shape[0]
    Bc = emb_ref.shape[1]
    wih = wih_ref[...]
    whh = whh_ref[...]
    bih = bih_ref[...]
    bhh = bhh_ref[...]

    def step(t, h):
        xt = emb_ref[t]
        gi = lax.dot_general(wih, xt, (((0,), (1,)), ((), ())),
                             preferred_element_type=jnp.float32) + bih
        gh = lax.dot_general(whh, h, (((0,), (0,)), ((), ())),
                             preferred_element_type=jnp.float32) + bhh
        i_r, i_z, i_n = gi[:Hd], gi[Hd:2 * Hd], gi[2 * Hd:]
        h_r, h_z, h_n = gh[:Hd], gh[Hd:2 * Hd], gh[2 * Hd:]
        r = jax.nn.sigmoid(i_r + h_r)
        z = jax.nn.sigmoid(i_z + h_z)
        n = jnp.tanh(i_n + r * h_n)
        h_new = (1.0 - z) * n + z * h
        outt_ref[t] = h_new
        return h_new

    h = jnp.zeros((Hd, Bc), jnp.float32)
    for t in range(Lx):
        h = step(t, h)


def _gru_t(emb, W_ih, W_hh, b_ih, b_hh, interpret=False):
    Lx, Bx, Hx = emb.shape
    wih_t = W_ih.T
    whh_t = W_hh.T
    bih2 = b_ih.reshape(3 * Hx, 1)
    bhh2 = b_hh.reshape(3 * Hx, 1)
    return pl.pallas_call(
        _gru_t_body,
        grid=(1,),
        in_specs=[
            pl.BlockSpec((Lx, Bx, Hx), lambda i: (0, 0, 0)),
            pl.BlockSpec((Hx, 3 * Hx), lambda i: (0, 0)),
            pl.BlockSpec((Hx, 3 * Hx), lambda i: (0, 0)),
            pl.BlockSpec((3 * Hx, 1), lambda i: (0, 0)),
            pl.BlockSpec((3 * Hx, 1), lambda i: (0, 0)),
        ],
        out_specs=pl.BlockSpec((Lx, Hx, Bx), lambda i: (0, 0, 0)),
        out_shape=jax.ShapeDtypeStruct((Lx, Hx, Bx), jnp.float32),
        interpret=interpret,
    )(emb, wih_t, whh_t, bih2, bhh2)


def kernel(x, E, W_ih, W_hh, b_ih, b_hh):
    Lx, Bx = x.shape
    Hx = E.shape[1]
    emb = _sc_gather(E, x.reshape(-1)).reshape(Lx, Bx, Hx)
    out_t = _gru_t(emb, W_ih, W_hh, b_ih, b_hh)
    out = jnp.transpose(out_t, (0, 2, 1))
    return out, out[Lx - 1:Lx]


# speedup vs baseline: 1.6871x; 1.0192x over previous
import functools

import jax
import jax.numpy as jnp
from jax import lax
from jax.experimental import pallas as pl
from jax.experimental.pallas import tpu as pltpu
from jax.experimental.pallas import tpu_sc as plsc


def _sc_gather(E, idx_flat):
    N = idx_flat.shape[0]
    D = E.shape[1]
    info = plsc.get_sparse_core_info()
    NC = info.num_cores
    NW = NC * info.num_subcores
    b_per_w = N // NW
    CH = 800
    n_chunks = b_per_w // CH
    assert N % NW == 0 and b_per_w % CH == 0 and CH % 16 == 0
    mesh = plsc.VectorSubcoreMesh(core_axis_name="c", subcore_axis_name="s")

    @functools.partial(
        pl.kernel,
        mesh=mesh,
        out_type=jax.ShapeDtypeStruct((N, D), jnp.float32),
        scratch_types=[
            pltpu.VMEM((CH,), jnp.int32),
            pltpu.VMEM((CH, D), jnp.float32),
            pltpu.SemaphoreType.DMA,
        ],
    )
    def gather_k(table_hbm, idx_hbm, out_hbm, idx_v, rows_v, sem):
        wid = lax.axis_index("s") * NC + lax.axis_index("c")
        base = wid * b_per_w

        def one_chunk(cbase):
            pltpu.sync_copy(idx_hbm.at[pl.ds(cbase, CH)], idx_v)

            @pl.loop(0, CH // 16)
            def _issue(g):
                v = idx_v[pl.ds(g * 16, 16)]
                for j in range(16):
                    pltpu.async_copy(
                        table_hbm.at[pl.ds(v[j], 1)],
                        rows_v.at[pl.ds(g * 16 + j, 1)],
                        sem,
                    )

                @pl.when(g > 2)
                def _():
                    for _ in range(16):
                        pltpu.make_async_copy(
                            table_hbm.at[pl.ds(0, 1)], rows_v.at[pl.ds(0, 1)], sem
                        ).wait()

            for _ in range(48):
                pltpu.make_async_copy(
                    table_hbm.at[pl.ds(0, 1)], rows_v.at[pl.ds(0, 1)], sem
                ).wait()

            pltpu.sync_copy(rows_v, out_hbm.at[pl.ds(cbase, CH)])

        for c in range(n_chunks):
            one_chunk(base + c * CH)

    return gather_k(E, idx_flat)


def _gru_t_body(emb_ref, wih_ref, whh_ref, bih_ref, bhh_ref, outt_ref):
    Lx = emb_ref.shape[0]
    Hd = wih_ref.shape[0]
    Bc = emb_ref.shape[1]
    wih = wih_ref[...]
    whh = whh_ref[...]
    bih = bih_ref[...]
    bhh = bhh_ref[...]

    def step(t, h):
        xt = emb_ref[t]
        gi = lax.dot_general(wih, xt, (((0,), (1,)), ((), ())),
                             preferred_element_type=jnp.float32) + bih
        gh = lax.dot_general(whh, h, (((0,), (0,)), ((), ())),
                             preferred_element_type=jnp.float32) + bhh
        i_r, i_z, i_n = gi[:Hd], gi[Hd:2 * Hd], gi[2 * Hd:]
        h_r, h_z, h_n = gh[:Hd], gh[Hd:2 * Hd], gh[2 * Hd:]
        r = jax.nn.sigmoid(i_r + h_r)
        z = jax.nn.sigmoid(i_z + h_z)
        n = jnp.tanh(i_n + r * h_n)
        h_new = (1.0 - z) * n + z * h
        outt_ref[t] = h_new
        return h_new

    h = jnp.zeros((Hd, Bc), jnp.float32)
    for t in range(Lx):
        h = step(t, h)


def _gru_t(emb, W_ih, W_hh, b_ih, b_hh, interpret=False):
    Lx, Bx, Hx = emb.shape
    wih_t = W_ih.T
    whh_t = W_hh.T
    bih2 = b_ih.reshape(3 * Hx, 1)
    bhh2 = b_hh.reshape(3 * Hx, 1)
    return pl.pallas_call(
        _gru_t_body,
        grid=(1,),
        in_specs=[
            pl.BlockSpec((Lx, Bx, Hx), lambda i: (0, 0, 0)),
            pl.BlockSpec((Hx, 3 * Hx), lambda i: (0, 0)),
            pl.BlockSpec((Hx, 3 * Hx), lambda i: (0, 0)),
            pl.BlockSpec((3 * Hx, 1), lambda i: (0, 0)),
            pl.BlockSpec((3 * Hx, 1), lambda i: (0, 0)),
        ],
        out_specs=pl.BlockSpec((Lx, Hx, Bx), lambda i: (0, 0, 0)),
        out_shape=jax.ShapeDtypeStruct((Lx, Hx, Bx), jnp.float32),
        interpret=interpret,
    )(emb, wih_t, whh_t, bih2, bhh2)


def kernel(x, E, W_ih, W_hh, b_ih, b_hh):
    Lx, Bx = x.shape
    Hx = E.shape[1]
    emb = _sc_gather(E, x.reshape(-1)).reshape(Lx, Bx, Hx)
    out_t = _gru_t(emb, W_ih, W_hh, b_ih, b_hh)
    out = jnp.transpose(out_t, (0, 2, 1))
    return out, out[Lx - 1:Lx]


# speedup vs baseline: 1.7006x; 1.0080x over previous
import functools

import jax
import jax.numpy as jnp
from jax import lax
from jax.experimental import pallas as pl
from jax.experimental.pallas import tpu as pltpu
from jax.experimental.pallas import tpu_sc as plsc


def _sc_gather(E, idx_flat):
    N = idx_flat.shape[0]
    D = E.shape[1]
    info = plsc.get_sparse_core_info()
    NC = info.num_cores
    NW = NC * info.num_subcores
    b_per_w = N // NW
    CH = 800
    n_chunks = b_per_w // CH
    assert N % NW == 0 and b_per_w % CH == 0 and CH % 16 == 0
    mesh = plsc.VectorSubcoreMesh(core_axis_name="c", subcore_axis_name="s")

    @functools.partial(
        pl.kernel,
        mesh=mesh,
        out_type=jax.ShapeDtypeStruct((N, D), jnp.float32),
        scratch_types=[
            pltpu.VMEM((CH,), jnp.int32),
            pltpu.VMEM((CH, D), jnp.float32),
            pltpu.SemaphoreType.DMA,
        ],
    )
    def gather_k(table_hbm, idx_hbm, out_hbm, idx_v, rows_v, sem):
        wid = lax.axis_index("s") * NC + lax.axis_index("c")
        base = wid * b_per_w

        def one_chunk(cbase):
            pltpu.sync_copy(idx_hbm.at[pl.ds(cbase, CH)], idx_v)

            @pl.loop(0, CH // 16)
            def _issue(g):
                v = idx_v[pl.ds(g * 16, 16)]
                for j in range(16):
                    pltpu.async_copy(
                        table_hbm.at[pl.ds(v[j], 1)],
                        rows_v.at[pl.ds(g * 16 + j, 1)],
                        sem,
                    )

                @pl.when(g > 3)
                def _():
                    for _ in range(16):
                        pltpu.make_async_copy(
                            table_hbm.at[pl.ds(0, 1)], rows_v.at[pl.ds(0, 1)], sem
                        ).wait()

            for _ in range(64):
                pltpu.make_async_copy(
                    table_hbm.at[pl.ds(0, 1)], rows_v.at[pl.ds(0, 1)], sem
                ).wait()

            pltpu.sync_copy(rows_v, out_hbm.at[pl.ds(cbase, CH)])

        for c in range(n_chunks):
            one_chunk(base + c * CH)

    return gather_k(E, idx_flat)


def _gru_t_body(emb_ref, wih_ref, whh_ref, bih_ref, bhh_ref, outt_ref):
    Lx = emb_ref.shape[0]
    Hd = wih_ref.shape[0]
    Bc = emb_ref.shape[1]
    wih = wih_ref[...]
    whh = whh_ref[...]
    bih = bih_ref[...]
    bhh = bhh_ref[...]

    def step(t, h):
        xt = emb_ref[t]
        gi = lax.dot_general(wih, xt, (((0,), (1,)), ((), ())),
                             preferred_element_type=jnp.float32) + bih
        gh = lax.dot_general(whh, h, (((0,), (0,)), ((), ())),
                             preferred_element_type=jnp.float32) + bhh
        i_r, i_z, i_n = gi[:Hd], gi[Hd:2 * Hd], gi[2 * Hd:]
        h_r, h_z, h_n = gh[:Hd], gh[Hd:2 * Hd], gh[2 * Hd:]
        r = jax.nn.sigmoid(i_r + h_r)
        z = jax.nn.sigmoid(i_z + h_z)
        n = jnp.tanh(i_n + r * h_n)
        h_new = (1.0 - z) * n + z * h
        outt_ref[t] = h_new
        return h_new

    h = jnp.zeros((Hd, Bc), jnp.float32)
    for t in range(Lx):
        h = step(t, h)


def _gru_t(emb, W_ih, W_hh, b_ih, b_hh, interpret=False):
    Lx, Bx, Hx = emb.shape
    wih_t = W_ih.T
    whh_t = W_hh.T
    bih2 = b_ih.reshape(3 * Hx, 1)
    bhh2 = b_hh.reshape(3 * Hx, 1)
    return pl.pallas_call(
        _gru_t_body,
        grid=(1,),
        in_specs=[
            pl.BlockSpec((Lx, Bx, Hx), lambda i: (0, 0, 0)),
            pl.BlockSpec((Hx, 3 * Hx), lambda i: (0, 0)),
            pl.BlockSpec((Hx, 3 * Hx), lambda i: (0, 0)),
            pl.BlockSpec((3 * Hx, 1), lambda i: (0, 0)),
            pl.BlockSpec((3 * Hx, 1), lambda i: (0, 0)),
        ],
        out_specs=pl.BlockSpec((Lx, Hx, Bx), lambda i: (0, 0, 0)),
        out_shape=jax.ShapeDtypeStruct((Lx, Hx, Bx), jnp.float32),
        interpret=interpret,
    )(emb, wih_t, whh_t, bih2, bhh2)


def kernel(x, E, W_ih, W_hh, b_ih, b_hh):
    Lx, Bx = x.shape
    Hx = E.shape[1]
    emb = _sc_gather(E, x.reshape(-1)).reshape(Lx, Bx, Hx)
    out_t = _gru_t(emb, W_ih, W_hh, b_ih, b_hh)
    out = jnp.transpose(out_t, (0, 2, 1))
    return out, out[Lx - 1:Lx]


# speedup vs baseline: 1.7379x; 1.0220x over previous
import functools

import jax
import jax.numpy as jnp
from jax import lax
from jax.experimental import pallas as pl
from jax.experimental.pallas import tpu as pltpu
from jax.experimental.pallas import tpu_sc as plsc


def _sc_gather(E, idx_flat):
    N = idx_flat.shape[0]
    D = E.shape[1]
    info = plsc.get_sparse_core_info()
    NC = info.num_cores
    NW = NC * info.num_subcores
    b_per_w = N // NW
    CH = 800
    n_chunks = b_per_w // CH
    assert N % NW == 0 and b_per_w % CH == 0 and CH % 16 == 0
    mesh = plsc.VectorSubcoreMesh(core_axis_name="c", subcore_axis_name="s")

    @functools.partial(
        pl.kernel,
        mesh=mesh,
        out_type=jax.ShapeDtypeStruct((N, D), jnp.float32),
        scratch_types=[
            pltpu.VMEM((CH,), jnp.int32),
            pltpu.VMEM((CH, D), jnp.float32),
            pltpu.SemaphoreType.DMA,
        ],
    )
    def gather_k(table_hbm, idx_hbm, out_hbm, idx_v, rows_v, sem):
        wid = lax.axis_index("s") * NC + lax.axis_index("c")
        base = wid * b_per_w

        def one_chunk(cbase):
            pltpu.sync_copy(idx_hbm.at[pl.ds(cbase, CH)], idx_v)

            @pl.loop(0, CH // 16)
            def _issue(g):
                v = idx_v[pl.ds(g * 16, 16)]
                for j in range(16):
                    pltpu.async_copy(
                        table_hbm.at[pl.ds(v[j], 1)],
                        rows_v.at[pl.ds(g * 16 + j, 1)],
                        sem,
                    )

                @pl.when(g > 3)
                def _():
                    for _ in range(16):
                        pltpu.make_async_copy(
                            table_hbm.at[pl.ds(0, 1)], rows_v.at[pl.ds(0, 1)], sem
                        ).wait()

            for _ in range(64):
                pltpu.make_async_copy(
                    table_hbm.at[pl.ds(0, 1)], rows_v.at[pl.ds(0, 1)], sem
                ).wait()

            pltpu.sync_copy(rows_v, out_hbm.at[pl.ds(cbase, CH)])

        for c in range(n_chunks):
            one_chunk(base + c * CH)

    return gather_k(E, idx_flat)


def _gru_t_body(emb_ref, wih_ref, whh_ref, bih_ref, bhh_ref, outt_ref, h_ref):
    TB = emb_ref.shape[0]
    Hd = wih_ref.shape[0]
    Bc = emb_ref.shape[1]
    wih = wih_ref[...]
    whh = whh_ref[...]
    bih = bih_ref[...]
    bhh = bhh_ref[...]

    @pl.when(pl.program_id(0) == 0)
    def _():
        h_ref[...] = jnp.zeros_like(h_ref)

    def step(t, h):
        xt = emb_ref[t]
        gi = lax.dot_general(wih, xt, (((0,), (1,)), ((), ())),
                             preferred_element_type=jnp.float32) + bih
        gh = lax.dot_general(whh, h, (((0,), (0,)), ((), ())),
                             preferred_element_type=jnp.float32) + bhh
        i_r, i_z, i_n = gi[:Hd], gi[Hd:2 * Hd], gi[2 * Hd:]
        h_r, h_z, h_n = gh[:Hd], gh[Hd:2 * Hd], gh[2 * Hd:]
        r = jax.nn.sigmoid(i_r + h_r)
        z = jax.nn.sigmoid(i_z + h_z)
        n = jnp.tanh(i_n + r * h_n)
        h_new = (1.0 - z) * n + z * h
        outt_ref[t] = h_new
        return h_new

    h = h_ref[...]
    for t in range(TB):
        h = step(t, h)
    h_ref[...] = h


def _gru_t(emb, W_ih, W_hh, b_ih, b_hh, interpret=False, tb=10):
    Lx, Bx, Hx = emb.shape
    wih_t = W_ih.T
    whh_t = W_hh.T
    bih2 = b_ih.reshape(3 * Hx, 1)
    bhh2 = b_hh.reshape(3 * Hx, 1)
    return pl.pallas_call(
        _gru_t_body,
        grid=(Lx // tb,),
        in_specs=[
            pl.BlockSpec((tb, Bx, Hx), lambda i: (i, 0, 0)),
            pl.BlockSpec((Hx, 3 * Hx), lambda i: (0, 0)),
            pl.BlockSpec((Hx, 3 * Hx), lambda i: (0, 0)),
            pl.BlockSpec((3 * Hx, 1), lambda i: (0, 0)),
            pl.BlockSpec((3 * Hx, 1), lambda i: (0, 0)),
        ],
        out_specs=pl.BlockSpec((tb, Hx, Bx), lambda i: (i, 0, 0)),
        out_shape=jax.ShapeDtypeStruct((Lx, Hx, Bx), jnp.float32),
        scratch_shapes=[pltpu.VMEM((Hx, Bx), jnp.float32)],
        interpret=interpret,
    )(emb, wih_t, whh_t, bih2, bhh2)


def kernel(x, E, W_ih, W_hh, b_ih, b_hh):
    Lx, Bx = x.shape
    Hx = E.shape[1]
    emb = _sc_gather(E, x.reshape(-1)).reshape(Lx, Bx, Hx)
    out_t = _gru_t(emb, W_ih, W_hh, b_ih, b_hh)
    out = jnp.transpose(out_t, (0, 2, 1))
    return out, out[Lx - 1:Lx]
